# baseline (speedup 1.0000x reference)
import functools

import jax
import jax.numpy as jnp
from jax.experimental import pallas as pl
from jax.experimental.pallas import tpu as pltpu

_HS = 768
_E = 64
_TOPK = 8
_BT = 512


def _router_body(x_ref, w_ref, wout_ref, iout_ref):
    x = x_ref[...]
    w = w_ref[...]
    logits = jax.lax.dot_general(
        x, w, (((1,), (1,)), ((), ())),
        preferred_element_type=jnp.float32)
    m = jnp.max(logits, axis=-1, keepdims=True)
    p = jnp.exp(logits - m)
    denom = jnp.sum(p, axis=-1, keepdims=True)

    lane = jax.lax.broadcasted_iota(jnp.int32, (_BT, _E), 1)
    kcol = jax.lax.broadcasted_iota(jnp.int32, (_BT, _TOPK), 1)
    wacc = jnp.zeros((_BT, _TOPK), jnp.float32)
    iacc = jnp.zeros((_BT, _TOPK), jnp.int32)
    vals = p
    for k in range(_TOPK):
        mk = jnp.max(vals, axis=-1, keepdims=True)
        idx = jnp.min(jnp.where(vals == mk, lane, _E),
                      axis=-1, keepdims=True)
        wacc = jnp.where(kcol == k, mk, wacc)
        iacc = jnp.where(kcol == k, idx, iacc)
        vals = jnp.where(lane == idx, -1.0, vals)
    wout_ref[...] = wacc / denom
    iout_ref[...] = iacc


@jax.jit
def _router(xf, W):
    n = xf.shape[0]
    grid = (n // _BT,)
    return pl.pallas_call(
        _router_body,
        grid=grid,
        in_specs=[
            pl.BlockSpec((_BT, _HS), lambda i: (i, 0)),
            pl.BlockSpec((_E, _HS), lambda i: (0, 0)),
        ],
        out_specs=[
            pl.BlockSpec((_BT, _TOPK), lambda i: (i, 0)),
            pl.BlockSpec((_BT, _TOPK), lambda i: (i, 0)),
        ],
        out_shape=[
            jax.ShapeDtypeStruct((n, _TOPK), jnp.float32),
            jax.ShapeDtypeStruct((n, _TOPK), jnp.int32),
        ],
    )(xf, W)


def kernel(x, W):
    xf = x.reshape(-1, x.shape[-1])
    w8, i8 = _router(xf, W)
    return (w8, i8)


# speedup vs baseline: 1.6234x; 1.6234x over previous
import functools

import jax
import jax.numpy as jnp
from jax.experimental import pallas as pl
from jax.experimental.pallas import tpu as pltpu

_HS = 768
_E = 64
_TOPK = 8
_BT = 512


def _router_body(x_ref, w_ref, wout_ref, iout_ref):
    x = x_ref[...]
    w = w_ref[...]
    logits = jax.lax.dot_general(
        w, x, (((1,), (1,)), ((), ())),
        preferred_element_type=jnp.float32)
    m = jnp.max(logits, axis=0, keepdims=True)
    p = jnp.exp(logits - m)
    rdenom = 1.0 / jnp.sum(p, axis=0, keepdims=True)

    eidx = jax.lax.broadcasted_iota(jnp.int32, (_E, _BT), 0)
    vals = p
    for k in range(_TOPK):
        mk = jnp.max(vals, axis=0, keepdims=True)
        hit = vals == mk
        idx = jnp.min(jnp.where(hit, eidx, _E), axis=0, keepdims=True)
        wout_ref[pl.ds(k, 1), :] = mk * rdenom
        iout_ref[pl.ds(k, 1), :] = idx
        vals = jnp.where(eidx == idx, -1.0, vals)


@jax.jit
def _router(xf, W):
    n = xf.shape[0]
    grid = (n // _BT,)
    return pl.pallas_call(
        _router_body,
        grid=grid,
        in_specs=[
            pl.BlockSpec((_BT, _HS), lambda i: (i, 0)),
            pl.BlockSpec((_E, _HS), lambda i: (0, 0)),
        ],
        out_specs=[
            pl.BlockSpec((_TOPK, _BT), lambda i: (0, i)),
            pl.BlockSpec((_TOPK, _BT), lambda i: (0, i)),
        ],
        out_shape=[
            jax.ShapeDtypeStruct((_TOPK, n), jnp.float32),
            jax.ShapeDtypeStruct((_TOPK, n), jnp.int32),
        ],
    )(xf, W)


def kernel(x, W):
    xf = x.reshape(-1, x.shape[-1])
    wT, iT = _router(xf, W)
    return (wT.T, iT.T)


# speedup vs baseline: 1.8421x; 1.1347x over previous
import functools

import jax
import jax.numpy as jnp
from jax.experimental import pallas as pl
from jax.experimental.pallas import tpu as pltpu

_HS = 768
_E = 64
_TOPK = 8
_BT = 1024


def _router_body(x_ref, w_ref, wout_ref, iout_ref):
    x = x_ref[...]
    w = w_ref[...]
    logits = jax.lax.dot_general(
        w, x, (((1,), (1,)), ((), ())),
        preferred_element_type=jnp.float32)
    m = jnp.max(logits, axis=0, keepdims=True)
    p = jnp.exp(logits - m)
    rdenom = 1.0 / jnp.sum(p, axis=0, keepdims=True)

    eidx = jax.lax.broadcasted_iota(jnp.int32, (_E, _BT), 0)
    vals = p
    for k in range(_TOPK):
        mk = jnp.max(vals, axis=0, keepdims=True)
        hit = vals == mk
        idx = jnp.min(jnp.where(hit, eidx, _E), axis=0, keepdims=True)
        wout_ref[pl.ds(k, 1), :] = mk * rdenom
        iout_ref[pl.ds(k, 1), :] = idx
        vals = jnp.where(eidx == idx, -1.0, vals)


@jax.jit
def _router(xf, W):
    n = xf.shape[0]
    grid = (n // _BT,)
    return pl.pallas_call(
        _router_body,
        grid=grid,
        in_specs=[
            pl.BlockSpec((_BT, _HS), lambda i: (i, 0)),
            pl.BlockSpec((_E, _HS), lambda i: (0, 0)),
        ],
        out_specs=[
            pl.BlockSpec((_TOPK, _BT), lambda i: (0, i)),
            pl.BlockSpec((_TOPK, _BT), lambda i: (0, i)),
        ],
        out_shape=[
            jax.ShapeDtypeStruct((_TOPK, n), jnp.float32),
            jax.ShapeDtypeStruct((_TOPK, n), jnp.int32),
        ],
    )(xf, W)


def kernel(x, W):
    xf = x.reshape(-1, x.shape[-1])
    wT, iT = _router(xf, W)
    return (wT.T, iT.T)


# speedup vs baseline: 1.9671x; 1.0678x over previous
import functools

import jax
import jax.numpy as jnp
from jax.experimental import pallas as pl
from jax.experimental.pallas import tpu as pltpu

_HS = 768
_E = 64
_TOPK = 8
_BT = 2048


def _router_body(x_ref, w_ref, wout_ref, iout_ref):
    x = x_ref[...]
    w = w_ref[...]
    logits = jax.lax.dot_general(
        w, x, (((1,), (1,)), ((), ())),
        preferred_element_type=jnp.float32)
    m = jnp.max(logits, axis=0, keepdims=True)
    p = jnp.exp(logits - m)
    rdenom = 1.0 / jnp.sum(p, axis=0, keepdims=True)

    eidx = jax.lax.broadcasted_iota(jnp.int32, (_E, _BT), 0)
    vals = p
    for k in range(_TOPK):
        mk = jnp.max(vals, axis=0, keepdims=True)
        hit = vals == mk
        idx = jnp.min(jnp.where(hit, eidx, _E), axis=0, keepdims=True)
        wout_ref[pl.ds(k, 1), :] = mk * rdenom
        iout_ref[pl.ds(k, 1), :] = idx
        vals = jnp.where(eidx == idx, -1.0, vals)


@jax.jit
def _router(xf, W):
    n = xf.shape[0]
    grid = (n // _BT,)
    return pl.pallas_call(
        _router_body,
        grid=grid,
        in_specs=[
            pl.BlockSpec((_BT, _HS), lambda i: (i, 0)),
            pl.BlockSpec((_E, _HS), lambda i: (0, 0)),
        ],
        out_specs=[
            pl.BlockSpec((_TOPK, _BT), lambda i: (0, i)),
            pl.BlockSpec((_TOPK, _BT), lambda i: (0, i)),
        ],
        out_shape=[
            jax.ShapeDtypeStruct((_TOPK, n), jnp.float32),
            jax.ShapeDtypeStruct((_TOPK, n), jnp.int32),
        ],
    )(xf, W)


def kernel(x, W):
    xf = x.reshape(-1, x.shape[-1])
    wT, iT = _router(xf, W)
    return (wT.T, iT.T)


# speedup vs baseline: 2.0003x; 1.0169x over previous
import functools

import jax
import jax.numpy as jnp
from jax.experimental import pallas as pl
from jax.experimental.pallas import tpu as pltpu

_HS = 768
_E = 64
_TOPK = 8
_BT = 4096


def _router_body(x_ref, w_ref, wout_ref, iout_ref):
    x = x_ref[...]
    w = w_ref[...]
    logits = jax.lax.dot_general(
        w, x, (((1,), (1,)), ((), ())),
        preferred_element_type=jnp.float32)
    m = jnp.max(logits, axis=0, keepdims=True)
    p = jnp.exp(logits - m)
    rdenom = 1.0 / jnp.sum(p, axis=0, keepdims=True)

    eidx = jax.lax.broadcasted_iota(jnp.int32, (_E, _BT), 0)
    vals = p
    for k in range(_TOPK):
        mk = jnp.max(vals, axis=0, keepdims=True)
        hit = vals == mk
        idx = jnp.min(jnp.where(hit, eidx, _E), axis=0, keepdims=True)
        wout_ref[pl.ds(k, 1), :] = mk * rdenom
        iout_ref[pl.ds(k, 1), :] = idx
        vals = jnp.where(eidx == idx, -1.0, vals)


@jax.jit
def _router(xf, W):
    n = xf.shape[0]
    grid = (n // _BT,)
    return pl.pallas_call(
        _router_body,
        grid=grid,
        in_specs=[
            pl.BlockSpec((_BT, _HS), lambda i: (i, 0)),
            pl.BlockSpec((_E, _HS), lambda i: (0, 0)),
        ],
        out_specs=[
            pl.BlockSpec((_TOPK, _BT), lambda i: (0, i)),
            pl.BlockSpec((_TOPK, _BT), lambda i: (0, i)),
        ],
        out_shape=[
            jax.ShapeDtypeStruct((_TOPK, n), jnp.float32),
            jax.ShapeDtypeStruct((_TOPK, n), jnp.int32),
        ],
    )(xf, W)


def kernel(x, W):
    xf = x.reshape(-1, x.shape[-1])
    wT, iT = _router(xf, W)
    return (wT.T, iT.T)
